# baseline (speedup 1.0000x reference)
import functools

import jax
import jax.numpy as jnp
from jax import lax
from jax.experimental import pallas as pl
from jax.experimental.pallas import tpu as pltpu
from jax.experimental.pallas import tpu_sc as plsc

N = 10000
D = 128
E = 320000
NEG = 0.2

NC = 2
NS = 16
NW = NC * NS
K = 128
NCHUNK = 80
EP = NW * K * NCHUNK
PN = N + 16
NP = 10240
RPT = NP // NS


def _attn_tail(h, asrc, adst):
    p = jnp.sum(h * asrc, axis=1)
    q = jnp.sum(h * adst, axis=1)
    t = p[:N] + q[:N]
    w0 = jnp.exp(jnp.where(t >= 0.0, t, NEG * t))
    return p, q, w0


def _pre_body(x_ref, w_ref, asrc_ref, adst_ref,
              h_ref, p_ref, q_ref, n0_ref, d0_ref):
    h = jnp.dot(x_ref[...], w_ref[...], preferred_element_type=jnp.float32)
    hp = jnp.concatenate([h, jnp.zeros((PN - N, D), jnp.float32)], axis=0)
    p, q, w0 = _attn_tail(hp, asrc_ref[...], adst_ref[...])
    h_ref[...] = hp
    p_ref[...] = p
    q_ref[...] = q
    n0_ref[...] = h * w0[:, None]
    d0_ref[...] = w0


def _mid_body(n0_ref, pa_ref, pb_ref, d0_ref, da_ref, db_ref, b_ref,
              w_ref, asrc_ref, adst_ref,
              h_ref, p_ref, q_ref, n0o_ref, d0o_ref):
    den = d0_ref[...] + da_ref[...] + db_ref[...] + 1e-16
    xr = (n0_ref[...] + pa_ref[...] + pb_ref[...]) / den[:, None] + b_ref[...]
    xr = jnp.maximum(xr, 0.0)
    h = jnp.dot(xr, w_ref[...], preferred_element_type=jnp.float32)
    hp = jnp.concatenate([h, jnp.zeros((PN - N, D), jnp.float32)], axis=0)
    p, q, w0 = _attn_tail(hp, asrc_ref[...], adst_ref[...])
    h_ref[...] = hp
    p_ref[...] = p
    q_ref[...] = q
    n0o_ref[...] = h * w0[:, None]
    d0o_ref[...] = w0


def _fin_body(n0_ref, pa_ref, pb_ref, d0_ref, da_ref, db_ref, b_ref, out_ref):
    den = d0_ref[...] + da_ref[...] + db_ref[...] + 1e-16
    out_ref[...] = (n0_ref[...] + pa_ref[...] + pb_ref[...]) / den[:, None] \
        + b_ref[...]


_OUT_MATP = jax.ShapeDtypeStruct((PN, D), jnp.float32)
_OUT_VECP = jax.ShapeDtypeStruct((PN,), jnp.float32)
_OUT_MAT = jax.ShapeDtypeStruct((N, D), jnp.float32)
_OUT_VEC = jax.ShapeDtypeStruct((N,), jnp.float32)


def _tc_pre(x, W, a_src, a_dst):
    return pl.pallas_call(
        _pre_body,
        out_shape=[_OUT_MATP, _OUT_VECP, _OUT_VECP, _OUT_MAT, _OUT_VEC],
    )(x, W, a_src.reshape(1, D), a_dst.reshape(1, D))


def _tc_mid(n0, pa, pb, d0, da, db, b, W, a_src, a_dst):
    return pl.pallas_call(
        _mid_body,
        out_shape=[_OUT_MATP, _OUT_VECP, _OUT_VECP, _OUT_MAT, _OUT_VEC],
    )(n0, pa, pb, d0, da, db, b.reshape(1, D), W,
      a_src.reshape(1, D), a_dst.reshape(1, D))


def _tc_fin(n0, pa, pb, d0, da, db, b):
    return pl.pallas_call(
        _fin_body,
        out_shape=_OUT_MAT,
    )(n0, pa, pb, d0, da, db, b.reshape(1, D))


_MESH = plsc.VectorSubcoreMesh(
    core_axis_name="c", subcore_axis_name="s", num_cores=NC, num_subcores=NS)


@functools.partial(
    pl.kernel,
    out_type=[jax.ShapeDtypeStruct((NC * NP, D), jnp.float32),
              jax.ShapeDtypeStruct((NC * NP,), jnp.float32)],
    mesh=_MESH,
    compiler_params=pltpu.CompilerParams(needs_layout_passes=False),
    scratch_types=[
        pltpu.VMEM((PN,), jnp.float32),
        pltpu.VMEM((PN,), jnp.float32),
        pltpu.VMEM((NCHUNK, K), jnp.int32),
        pltpu.VMEM((K, D), jnp.float32),
        pltpu.VMEM((K,), jnp.int32),
        pltpu.VMEM((K,), jnp.int32),
        pltpu.VMEM((K,), jnp.float32),
        pltpu.VMEM_SHARED((NP, D), jnp.float32),
        pltpu.VMEM_SHARED((NP,), jnp.float32),
        pltpu.SemaphoreType.DMA,
        pltpu.SemaphoreType.DMA,
        pltpu.SemaphoreType.DMA,
    ],
)
def _sc_edge(src2d_hbm, dst1d_hbm, h_hbm, p_hbm, q_hbm, z_hbm, zn_hbm,
             parts_hbm, dparts_hbm,
             p_loc, q_loc, src2, rows, dst_c0, dst_c1, w_c,
             numer_sh, den_sh, semr, semi0, semi1):
    c = lax.axis_index("c")
    s = lax.axis_index("s")
    wid = c * NS + s

    pltpu.sync_copy(z_hbm.at[pl.ds(s * RPT, RPT)],
                    numer_sh.at[pl.ds(s * RPT, RPT)])
    @pl.when(s == 0)
    def _():
        pltpu.sync_copy(zn_hbm, den_sh)
    pltpu.sync_copy(p_hbm, p_loc)
    pltpu.sync_copy(q_hbm, q_loc)
    pltpu.sync_copy(src2d_hbm.at[pl.ds(wid * NCHUNK, NCHUNK)], src2)

    plsc.subcore_barrier()

    def start_dst(ci, dst_c, semi):
        pltpu.async_copy(
            dst1d_hbm.at[pl.ds((wid * NCHUNK + ci) * K, K)], dst_c, semi)

    def do_chunk(ci, dst_c, semi):
        pltpu.async_copy(h_hbm.at[src2.at[ci]], rows, semr)
        pltpu.make_async_copy(
            dst1d_hbm.at[pl.ds((wid * NCHUNK + ci) * K, K)], dst_c,
            semi).wait()

        def w_body(j, carry):
            sv = src2[ci, pl.ds(j * 16, 16)]
            dv = dst_c[pl.ds(j * 16, 16)]
            tt = plsc.load_gather(p_loc, [sv]) + plsc.load_gather(q_loc, [dv])
            tt = jnp.where(tt >= 0.0, tt, NEG * tt)
            w_c[pl.ds(j * 16, 16)] = jnp.exp(tt)
            return carry
        lax.fori_loop(0, K // 16, w_body, 0)

        pltpu.make_async_copy(h_hbm.at[src2.at[ci]], rows, semr).wait()

        def r_body(e, carry):
            wb = plsc.load_gather(w_c, [jnp.full((16,), e, jnp.int32)])
            for j in range(D // 16):
                rows[e, pl.ds(j * 16, 16)] = rows[e, pl.ds(j * 16, 16)] * wb
            return carry
        lax.fori_loop(0, K, r_body, 0)

        pltpu.sync_copy(rows, numer_sh.at[dst_c], add=True)
        pltpu.sync_copy(w_c, den_sh.at[dst_c], add=True)

        @pl.when(ci + 2 < NCHUNK)
        def _():
            start_dst(ci + 2, dst_c, semi)

    start_dst(0, dst_c0, semi0)
    start_dst(1, dst_c1, semi1)

    def pipe_body(i, carry):
        do_chunk(2 * i, dst_c0, semi0)
        do_chunk(2 * i + 1, dst_c1, semi1)
        return carry
    lax.fori_loop(0, NCHUNK // 2, pipe_body, 0)

    plsc.subcore_barrier()
    pltpu.sync_copy(numer_sh.at[pl.ds(s * RPT, RPT)],
                    parts_hbm.at[pl.ds(c * NP + s * RPT, RPT)])
    @pl.when(s == 0)
    def _():
        pltpu.sync_copy(den_sh, dparts_hbm.at[pl.ds(c * NP, NP)])


def kernel(x, edge_index, W1, a_src1, a_dst1, b1, W2, a_src2, a_dst2, b2):
    pad = jnp.full((EP - E,), N, jnp.int32)
    src2d = jnp.concatenate([edge_index[0], pad]).reshape(EP // K, K)
    dst1d = jnp.concatenate([edge_index[1], pad])
    zrow = jnp.zeros((NP, D), jnp.float32)
    zn = jnp.zeros((NP,), jnp.float32)

    h1, p1, q1, n01, d01 = _tc_pre(x, W1, a_src1, a_dst1)
    parts1, dparts1 = _sc_edge(src2d, dst1d, h1, p1, q1, zrow, zn)
    h2, p2, q2, n02, d02 = _tc_mid(
        n01, parts1[:N], parts1[NP:NP + N], d01, dparts1[:N],
        dparts1[NP:NP + N], b1, W2, a_src2, a_dst2)
    parts2, dparts2 = _sc_edge(src2d, dst1d, h2, p2, q2, zrow, zn)
    out = _tc_fin(n02, parts2[:N], parts2[NP:NP + N], d02, dparts2[:N],
                  dparts2[NP:NP + N], b2)
    return out


# speedup vs baseline: 1.1148x; 1.1148x over previous
import functools

import jax
import jax.numpy as jnp
from jax import lax
from jax.experimental import pallas as pl
from jax.experimental.pallas import tpu as pltpu
from jax.experimental.pallas import tpu_sc as plsc

N = 10000
D = 128
E = 320000
NEG = 0.2

NC = 2
NS = 16
NW = NC * NS
K = 64
NCHUNK = 160
EP = NW * K * NCHUNK
PN = N + 16
NP = 10112
RPT = NP // NS


def _attn_tail(h, asrc, adst):
    p = jnp.sum(h * asrc, axis=1)
    q = jnp.sum(h * adst, axis=1)
    t = p[:N] + q[:N]
    w0 = jnp.exp(jnp.where(t >= 0.0, t, NEG * t))
    return p, q, w0


def _pre_body(x_ref, w_ref, asrc_ref, adst_ref,
              h_ref, p_ref, q_ref, n0_ref, d0_ref):
    h = jnp.dot(x_ref[...], w_ref[...], preferred_element_type=jnp.float32)
    hp = jnp.concatenate([h, jnp.zeros((PN - N, D), jnp.float32)], axis=0)
    p, q, w0 = _attn_tail(hp, asrc_ref[...], adst_ref[...])
    h_ref[...] = hp
    p_ref[...] = p
    q_ref[...] = q
    n0_ref[...] = h * w0[:, None]
    d0_ref[...] = w0


def _mid_body(n0_ref, pa_ref, pb_ref, d0_ref, da_ref, db_ref, b_ref,
              w_ref, asrc_ref, adst_ref,
              h_ref, p_ref, q_ref, n0o_ref, d0o_ref):
    den = d0_ref[...] + da_ref[...] + db_ref[...] + 1e-16
    xr = (n0_ref[...] + pa_ref[...] + pb_ref[...]) / den[:, None] + b_ref[...]
    xr = jnp.maximum(xr, 0.0)
    h = jnp.dot(xr, w_ref[...], preferred_element_type=jnp.float32)
    hp = jnp.concatenate([h, jnp.zeros((PN - N, D), jnp.float32)], axis=0)
    p, q, w0 = _attn_tail(hp, asrc_ref[...], adst_ref[...])
    h_ref[...] = hp
    p_ref[...] = p
    q_ref[...] = q
    n0o_ref[...] = h * w0[:, None]
    d0o_ref[...] = w0


def _fin_body(n0_ref, pa_ref, pb_ref, d0_ref, da_ref, db_ref, b_ref, out_ref):
    den = d0_ref[...] + da_ref[...] + db_ref[...] + 1e-16
    out_ref[...] = (n0_ref[...] + pa_ref[...] + pb_ref[...]) / den[:, None] \
        + b_ref[...]


_OUT_MATP = jax.ShapeDtypeStruct((PN, D), jnp.float32)
_OUT_VECP = jax.ShapeDtypeStruct((PN,), jnp.float32)
_OUT_MAT = jax.ShapeDtypeStruct((N, D), jnp.float32)
_OUT_VEC = jax.ShapeDtypeStruct((N,), jnp.float32)


def _tc_pre(x, W, a_src, a_dst):
    return pl.pallas_call(
        _pre_body,
        out_shape=[_OUT_MATP, _OUT_VECP, _OUT_VECP, _OUT_MAT, _OUT_VEC],
    )(x, W, a_src.reshape(1, D), a_dst.reshape(1, D))


def _tc_mid(n0, pa, pb, d0, da, db, b, W, a_src, a_dst):
    return pl.pallas_call(
        _mid_body,
        out_shape=[_OUT_MATP, _OUT_VECP, _OUT_VECP, _OUT_MAT, _OUT_VEC],
    )(n0, pa, pb, d0, da, db, b.reshape(1, D), W,
      a_src.reshape(1, D), a_dst.reshape(1, D))


def _tc_fin(n0, pa, pb, d0, da, db, b):
    return pl.pallas_call(
        _fin_body,
        out_shape=_OUT_MAT,
    )(n0, pa, pb, d0, da, db, b.reshape(1, D))


_MESH = plsc.VectorSubcoreMesh(
    core_axis_name="c", subcore_axis_name="s", num_cores=NC, num_subcores=NS)


@functools.partial(
    pl.kernel,
    out_type=[jax.ShapeDtypeStruct((NC * NP, D), jnp.float32),
              jax.ShapeDtypeStruct((NC * NP,), jnp.float32)],
    mesh=_MESH,
    compiler_params=pltpu.CompilerParams(needs_layout_passes=False),
    scratch_types=[
        pltpu.VMEM((PN,), jnp.float32),
        pltpu.VMEM((PN,), jnp.float32),
        pltpu.VMEM((K, D), jnp.float32),
        pltpu.VMEM((K, D), jnp.float32),
        pltpu.VMEM((K,), jnp.int32),
        pltpu.VMEM((K,), jnp.int32),
        pltpu.VMEM((K,), jnp.int32),
        pltpu.VMEM((K,), jnp.int32),
        pltpu.VMEM((K,), jnp.float32),
        pltpu.VMEM((K,), jnp.float32),
        pltpu.VMEM_SHARED((NP, D), jnp.float32),
        pltpu.VMEM_SHARED((NP,), jnp.float32),
        pltpu.SemaphoreType.DMA,
        pltpu.SemaphoreType.DMA,
        pltpu.SemaphoreType.DMA,
        pltpu.SemaphoreType.DMA,
        pltpu.SemaphoreType.DMA,
        pltpu.SemaphoreType.DMA,
        pltpu.SemaphoreType.DMA,
        pltpu.SemaphoreType.DMA,
        pltpu.SemaphoreType.DMA,
        pltpu.SemaphoreType.DMA,
    ],
)
def _sc_edge(srcA_hbm, srcB_hbm, dstA_hbm, dstB_hbm, h_hbm, p_hbm, q_hbm,
             z_hbm, zn_hbm, parts_hbm, dparts_hbm,
             p_loc, q_loc, rows0, rows1, src_c0, src_c1, dst_c0, dst_c1,
             w_c0, w_c1, numer_sh, den_sh, semr0, semr1, semsrc0, semsrc1,
             semi0, semi1, sems0, sems1, semd0, semd1):
    c = lax.axis_index("c")
    s = lax.axis_index("s")
    wid = c * NS + s

    pltpu.sync_copy(z_hbm.at[pl.ds(s * RPT, RPT)],
                    numer_sh.at[pl.ds(s * RPT, RPT)])
    @pl.when(s == 0)
    def _():
        pltpu.sync_copy(zn_hbm, den_sh)
    pltpu.sync_copy(p_hbm, p_loc)
    pltpu.sync_copy(q_hbm, q_loc)

    plsc.subcore_barrier()

    def idx_view(arrA, arrB, ci, odd):
        g = (wid * NCHUNK + ci) * K
        if odd:
            return arrB.at[pl.ds(g - K, K)]
        return arrA.at[pl.ds(g, K)]

    def fetch_src(ci, odd, src_c, semsrc):
        pltpu.async_copy(idx_view(srcA_hbm, srcB_hbm, ci, odd), src_c,
                         semsrc)

    def fetch_dst(ci, odd, dst_c, semi):
        pltpu.async_copy(idx_view(dstA_hbm, dstB_hbm, ci, odd), dst_c, semi)

    def start_gather(ci, odd, rows, src_c, semr, semsrc):
        pltpu.make_async_copy(idx_view(srcA_hbm, srcB_hbm, ci, odd), src_c,
                              semsrc).wait()
        pltpu.async_copy(h_hbm.at[src_c], rows, semr)

    def work_chunk(ci, odd, rows, src_c, dst_c, w_c,
                   semr, semsrc, semi, sems, semd):
        pltpu.make_async_copy(idx_view(dstA_hbm, dstB_hbm, ci, odd), dst_c,
                              semi).wait()

        def w_body(j, carry):
            sv = src_c[pl.ds(j * 16, 16)]
            dv = dst_c[pl.ds(j * 16, 16)]
            tt = plsc.load_gather(p_loc, [sv]) + plsc.load_gather(q_loc, [dv])
            tt = jnp.where(tt >= 0.0, tt, NEG * tt)
            w_c[pl.ds(j * 16, 16)] = jnp.exp(tt)
            return carry
        lax.fori_loop(0, K // 16, w_body, 0)

        pltpu.make_async_copy(h_hbm.at[src_c], rows, semr).wait()

        @pl.when(ci + 2 < NCHUNK)
        def _():
            fetch_src(ci + 2, odd, src_c, semsrc)

        def r_body(e, carry):
            wb = plsc.load_gather(w_c, [jnp.full((16,), e, jnp.int32)])
            for j in range(D // 16):
                rows[e, pl.ds(j * 16, 16)] = rows[e, pl.ds(j * 16, 16)] * wb
            return carry
        lax.fori_loop(0, K, r_body, 0)

        pltpu.async_copy(rows, numer_sh.at[dst_c], sems, add=True)
        pltpu.async_copy(w_c, den_sh.at[dst_c], semd, add=True)

    def drain_chunk(rows, dst_c, w_c, sems, semd):
        pltpu.make_async_copy(rows, numer_sh.at[dst_c], sems).wait()
        pltpu.make_async_copy(w_c, den_sh.at[dst_c], semd).wait()

    fetch_src(0, False, src_c0, semsrc0)
    fetch_dst(0, False, dst_c0, semi0)
    fetch_src(1, True, src_c1, semsrc1)
    fetch_dst(1, True, dst_c1, semi1)
    start_gather(0, False, rows0, src_c0, semr0, semsrc0)
    start_gather(1, True, rows1, src_c1, semr1, semsrc1)

    def pipe_body(i, carry):
        ci = 2 * i
        work_chunk(ci, False, rows0, src_c0, dst_c0, w_c0,
                   semr0, semsrc0, semi0, sems0, semd0)
        work_chunk(ci + 1, True, rows1, src_c1, dst_c1, w_c1,
                   semr1, semsrc1, semi1, sems1, semd1)
        drain_chunk(rows0, dst_c0, w_c0, sems0, semd0)
        @pl.when(ci + 2 < NCHUNK)
        def _():
            fetch_dst(ci + 2, False, dst_c0, semi0)
            start_gather(ci + 2, False, rows0, src_c0, semr0, semsrc0)
        drain_chunk(rows1, dst_c1, w_c1, sems1, semd1)
        @pl.when(ci + 3 < NCHUNK)
        def _():
            fetch_dst(ci + 3, True, dst_c1, semi1)
            start_gather(ci + 3, True, rows1, src_c1, semr1, semsrc1)
        return carry
    lax.fori_loop(0, NCHUNK // 2, pipe_body, 0)

    plsc.subcore_barrier()
    pltpu.sync_copy(numer_sh.at[pl.ds(s * RPT, RPT)],
                    parts_hbm.at[pl.ds(c * NP + s * RPT, RPT)])
    @pl.when(s == 0)
    def _():
        pltpu.sync_copy(den_sh, dparts_hbm.at[pl.ds(c * NP, NP)])


def kernel(x, edge_index, W1, a_src1, a_dst1, b1, W2, a_src2, a_dst2, b2):
    pad = jnp.full((EP - E,), N, jnp.int32)
    padK = jnp.full((K,), N, jnp.int32)
    srcA = jnp.concatenate([edge_index[0], pad])
    srcB = jnp.concatenate([srcA[K:], padK])
    dstA = jnp.concatenate([edge_index[1], pad])
    dstB = jnp.concatenate([dstA[K:], padK])
    zrow = jnp.zeros((NP, D), jnp.float32)
    zn = jnp.zeros((NP,), jnp.float32)

    h1, p1, q1, n01, d01 = _tc_pre(x, W1, a_src1, a_dst1)
    parts1, dparts1 = _sc_edge(srcA, srcB, dstA, dstB, h1, p1, q1, zrow, zn)
    h2, p2, q2, n02, d02 = _tc_mid(
        n01, parts1[:N], parts1[NP:NP + N], d01, dparts1[:N],
        dparts1[NP:NP + N], b1, W2, a_src2, a_dst2)
    parts2, dparts2 = _sc_edge(srcA, srcB, dstA, dstB, h2, p2, q2, zrow, zn)
    out = _tc_fin(n02, parts2[:N], parts2[NP:NP + N], d02, dparts2[:N],
                  dparts2[NP:NP + N], b2)
    return out


# speedup vs baseline: 1.1850x; 1.0630x over previous
import functools

import jax
import jax.numpy as jnp
from jax import lax
from jax.experimental import pallas as pl
from jax.experimental.pallas import tpu as pltpu
from jax.experimental.pallas import tpu_sc as plsc

N = 10000
D = 128
E = 320000
NEG = 0.2

NC = 2
NS = 16
NW = NC * NS
K = 64
NCHUNK = 160
EP = NW * K * NCHUNK
NP = 10112
PN = NP
RPT = NP // NS


def _attn_tail(h, asrc, adst):
    p = jnp.sum(h * asrc, axis=1)
    q = jnp.sum(h * adst, axis=1)
    t = p[:N] + q[:N]
    w0 = jnp.exp(jnp.where(t >= 0.0, t, NEG * t))
    return p, q, w0


def _pre_body(x_ref, w_ref, asrc_ref, adst_ref,
              h_ref, p_ref, q_ref, n0_ref, d0_ref):
    h = jnp.dot(x_ref[...], w_ref[...], preferred_element_type=jnp.float32)
    hp = jnp.concatenate([h, jnp.zeros((PN - N, D), jnp.float32)], axis=0)
    p, q, w0 = _attn_tail(hp, asrc_ref[...], adst_ref[...])
    h_ref[...] = hp
    p_ref[...] = p
    q_ref[...] = q
    n0_ref[...] = h * w0[:, None]
    d0_ref[...] = w0


def _mid_body(n0_ref, pa_ref, pb_ref, d0_ref, da_ref, db_ref, b_ref,
              w_ref, asrc_ref, adst_ref,
              h_ref, p_ref, q_ref, n0o_ref, d0o_ref):
    den = d0_ref[...] + da_ref[...] + db_ref[...] + 1e-16
    xr = (n0_ref[...] + pa_ref[...] + pb_ref[...]) / den[:, None] + b_ref[...]
    xr = jnp.maximum(xr, 0.0)
    h = jnp.dot(xr, w_ref[...], preferred_element_type=jnp.float32)
    hp = jnp.concatenate([h, jnp.zeros((PN - N, D), jnp.float32)], axis=0)
    p, q, w0 = _attn_tail(hp, asrc_ref[...], adst_ref[...])
    h_ref[...] = hp
    p_ref[...] = p
    q_ref[...] = q
    n0o_ref[...] = h * w0[:, None]
    d0o_ref[...] = w0


def _fin_body(n0_ref, pa_ref, pb_ref, d0_ref, da_ref, db_ref, b_ref, out_ref):
    den = d0_ref[...] + da_ref[...] + db_ref[...] + 1e-16
    out_ref[...] = (n0_ref[...] + pa_ref[...] + pb_ref[...]) / den[:, None] \
        + b_ref[...]


_OUT_MATP = jax.ShapeDtypeStruct((PN, D), jnp.float32)
_OUT_VECP = jax.ShapeDtypeStruct((PN,), jnp.float32)
_OUT_MAT = jax.ShapeDtypeStruct((N, D), jnp.float32)
_OUT_VEC = jax.ShapeDtypeStruct((N,), jnp.float32)


def _tc_pre(x, W, a_src, a_dst):
    return pl.pallas_call(
        _pre_body,
        out_shape=[_OUT_MATP, _OUT_VECP, _OUT_VECP, _OUT_MAT, _OUT_VEC],
    )(x, W, a_src.reshape(1, D), a_dst.reshape(1, D))


def _tc_mid(n0, pa, pb, d0, da, db, b, W, a_src, a_dst):
    return pl.pallas_call(
        _mid_body,
        out_shape=[_OUT_MATP, _OUT_VECP, _OUT_VECP, _OUT_MAT, _OUT_VEC],
    )(n0, pa, pb, d0, da, db, b.reshape(1, D), W,
      a_src.reshape(1, D), a_dst.reshape(1, D))


def _tc_fin(n0, pa, pb, d0, da, db, b):
    return pl.pallas_call(
        _fin_body,
        out_shape=_OUT_MAT,
    )(n0, pa, pb, d0, da, db, b.reshape(1, D))


_MESH = plsc.VectorSubcoreMesh(
    core_axis_name="c", subcore_axis_name="s", num_cores=NC, num_subcores=NS)


@functools.partial(
    pl.kernel,
    out_type=[jax.ShapeDtypeStruct((NC * NP, D), jnp.float32),
              jax.ShapeDtypeStruct((NC * NP,), jnp.float32)],
    mesh=_MESH,
    compiler_params=pltpu.CompilerParams(needs_layout_passes=False),
    scratch_types=[
        pltpu.VMEM((PN,), jnp.float32),
        pltpu.VMEM((PN,), jnp.float32),
        pltpu.VMEM((K, D), jnp.float32),
        pltpu.VMEM((K, D), jnp.float32),
        pltpu.VMEM((K,), jnp.int32),
        pltpu.VMEM((K,), jnp.int32),
        pltpu.VMEM((K,), jnp.int32),
        pltpu.VMEM((K,), jnp.int32),
        pltpu.VMEM((K,), jnp.float32),
        pltpu.VMEM((K,), jnp.float32),
        pltpu.VMEM_SHARED((NP, D), jnp.float32),
        pltpu.VMEM_SHARED((NP,), jnp.float32),
        pltpu.SemaphoreType.DMA,
        pltpu.SemaphoreType.DMA,
        pltpu.SemaphoreType.DMA,
        pltpu.SemaphoreType.DMA,
        pltpu.SemaphoreType.DMA,
        pltpu.SemaphoreType.DMA,
        pltpu.SemaphoreType.DMA,
        pltpu.SemaphoreType.DMA,
        pltpu.SemaphoreType.DMA,
        pltpu.SemaphoreType.DMA,
    ],
)
def _sc_edge(srcA_hbm, srcB_hbm, dstA_hbm, dstB_hbm, h_hbm, p_hbm, q_hbm,
             z_hbm, zn_hbm, parts_hbm, dparts_hbm,
             p_loc, q_loc, rows0, rows1, src_c0, src_c1, dst_c0, dst_c1,
             w_c0, w_c1, numer_sh, den_sh, semr0, semr1, semsrc0, semsrc1,
             semi0, semi1, sems0, sems1, semd0, semd1):
    c = lax.axis_index("c")
    s = lax.axis_index("s")
    wid = c * NS + s

    pltpu.sync_copy(z_hbm.at[pl.ds(s * RPT, RPT)],
                    numer_sh.at[pl.ds(s * RPT, RPT)])
    @pl.when(s == 0)
    def _():
        pltpu.sync_copy(zn_hbm, den_sh)
    pltpu.sync_copy(p_hbm, p_loc)
    pltpu.sync_copy(q_hbm, q_loc)

    plsc.subcore_barrier()

    def idx_view(arrA, arrB, ci, odd):
        g = (wid * NCHUNK + ci) * K
        if odd:
            return arrB.at[pl.ds(g - K, K)]
        return arrA.at[pl.ds(g, K)]

    def fetch_src(ci, odd, src_c, semsrc):
        pltpu.async_copy(idx_view(srcA_hbm, srcB_hbm, ci, odd), src_c,
                         semsrc)

    def fetch_dst(ci, odd, dst_c, semi):
        pltpu.async_copy(idx_view(dstA_hbm, dstB_hbm, ci, odd), dst_c, semi)

    def start_gather(ci, odd, rows, src_c, semr, semsrc):
        pltpu.make_async_copy(idx_view(srcA_hbm, srcB_hbm, ci, odd), src_c,
                              semsrc).wait()
        pltpu.async_copy(h_hbm.at[src_c], rows, semr)

    def work_chunk(ci, odd, rows, src_c, dst_c, w_c,
                   semr, semsrc, semi, sems, semd):
        pltpu.make_async_copy(idx_view(dstA_hbm, dstB_hbm, ci, odd), dst_c,
                              semi).wait()

        def w_body(j, carry):
            sv = src_c[pl.ds(j * 16, 16)]
            dv = dst_c[pl.ds(j * 16, 16)]
            tt = plsc.load_gather(p_loc, [sv]) + plsc.load_gather(q_loc, [dv])
            tt = jnp.where(tt >= 0.0, tt, NEG * tt)
            w_c[pl.ds(j * 16, 16)] = jnp.exp(tt)
            return carry
        lax.fori_loop(0, K // 16, w_body, 0)

        pltpu.make_async_copy(h_hbm.at[src_c], rows, semr).wait()

        @pl.when(ci + 2 < NCHUNK)
        def _():
            fetch_src(ci + 2, odd, src_c, semsrc)

        def r_body(e, carry):
            wb = plsc.load_gather(w_c, [jnp.full((16,), e, jnp.int32)])
            for j in range(D // 16):
                rows[e, pl.ds(j * 16, 16)] = rows[e, pl.ds(j * 16, 16)] * wb
            return carry
        lax.fori_loop(0, K, r_body, 0)

        pltpu.async_copy(rows, numer_sh.at[dst_c], sems, add=True)
        pltpu.async_copy(w_c, den_sh.at[dst_c], semd, add=True)

    def drain_chunk(rows, dst_c, w_c, sems, semd):
        pltpu.make_async_copy(rows, numer_sh.at[dst_c], sems).wait()
        pltpu.make_async_copy(w_c, den_sh.at[dst_c], semd).wait()

    fetch_src(0, False, src_c0, semsrc0)
    fetch_dst(0, False, dst_c0, semi0)
    fetch_src(1, True, src_c1, semsrc1)
    fetch_dst(1, True, dst_c1, semi1)
    start_gather(0, False, rows0, src_c0, semr0, semsrc0)
    start_gather(1, True, rows1, src_c1, semr1, semsrc1)

    def pipe_body(i, carry):
        ci = 2 * i
        work_chunk(ci, False, rows0, src_c0, dst_c0, w_c0,
                   semr0, semsrc0, semi0, sems0, semd0)
        work_chunk(ci + 1, True, rows1, src_c1, dst_c1, w_c1,
                   semr1, semsrc1, semi1, sems1, semd1)
        drain_chunk(rows0, dst_c0, w_c0, sems0, semd0)
        @pl.when(ci + 2 < NCHUNK)
        def _():
            fetch_dst(ci + 2, False, dst_c0, semi0)
            start_gather(ci + 2, False, rows0, src_c0, semr0, semsrc0)
        drain_chunk(rows1, dst_c1, w_c1, sems1, semd1)
        @pl.when(ci + 3 < NCHUNK)
        def _():
            fetch_dst(ci + 3, True, dst_c1, semi1)
            start_gather(ci + 3, True, rows1, src_c1, semr1, semsrc1)
        return carry
    lax.fori_loop(0, NCHUNK // 2, pipe_body, 0)

    plsc.subcore_barrier()
    pltpu.sync_copy(numer_sh.at[pl.ds(s * RPT, RPT)],
                    parts_hbm.at[pl.ds(c * NP + s * RPT, RPT)])
    @pl.when(s == 0)
    def _():
        pltpu.sync_copy(den_sh, dparts_hbm.at[pl.ds(c * NP, NP)])


def kernel(x, edge_index, W1, a_src1, a_dst1, b1, W2, a_src2, a_dst2, b2):
    pad = jnp.full((EP - E,), N, jnp.int32)
    padK = jnp.full((K,), N, jnp.int32)
    padd = N + jnp.arange(EP - E, dtype=jnp.int32) % (NP - N)
    srcA = jnp.concatenate([edge_index[0], pad])
    srcB = jnp.concatenate([srcA[K:], padK])
    dstA = jnp.concatenate([edge_index[1], padd])
    dstB = jnp.concatenate([dstA[K:], padK])
    zrow = jnp.zeros((NP, D), jnp.float32)
    zn = jnp.zeros((NP,), jnp.float32)

    h1, p1, q1, n01, d01 = _tc_pre(x, W1, a_src1, a_dst1)
    parts1, dparts1 = _sc_edge(srcA, srcB, dstA, dstB, h1, p1, q1, zrow, zn)
    h2, p2, q2, n02, d02 = _tc_mid(
        n01, parts1[:N], parts1[NP:NP + N], d01, dparts1[:N],
        dparts1[NP:NP + N], b1, W2, a_src2, a_dst2)
    parts2, dparts2 = _sc_edge(srcA, srcB, dstA, dstB, h2, p2, q2, zrow, zn)
    out = _tc_fin(n02, parts2[:N], parts2[NP:NP + N], d02, dparts2[:N],
                  dparts2[NP:NP + N], b2)
    return out


# speedup vs baseline: 1.1877x; 1.0023x over previous
import functools

import jax
import jax.numpy as jnp
from jax import lax
from jax.experimental import pallas as pl
from jax.experimental.pallas import tpu as pltpu
from jax.experimental.pallas import tpu_sc as plsc

N = 10000
D = 128
E = 320000
NEG = 0.2

NC = 2
NS = 16
NW = NC * NS
K = 64
NCHUNK = 160
EP = NW * K * NCHUNK
NP = 10112
PN = NP
RPT = NP // NS


def _attn_tail(h, asrc, adst):
    p = jnp.sum(h * asrc, axis=1)
    q = jnp.sum(h * adst, axis=1)
    t = p[:N] + q[:N]
    w0 = jnp.exp(jnp.where(t >= 0.0, t, NEG * t))
    return p, q, w0


def _pre_body(x_ref, w_ref, asrc_ref, adst_ref,
              h_ref, p_ref, q_ref, n0_ref, d0_ref):
    h = jnp.dot(x_ref[...], w_ref[...], preferred_element_type=jnp.float32)
    hp = jnp.concatenate([h, jnp.zeros((PN - N, D), jnp.float32)], axis=0)
    p, q, w0 = _attn_tail(hp, asrc_ref[...], adst_ref[...])
    h_ref[...] = hp
    p_ref[...] = p
    q_ref[...] = q
    n0_ref[...] = h * w0[:, None]
    d0_ref[...] = w0


def _mid_body(n0_ref, pa_ref, pb_ref, d0_ref, da_ref, db_ref, b_ref,
              w_ref, asrc_ref, adst_ref,
              h_ref, p_ref, q_ref, n0o_ref, d0o_ref):
    den = d0_ref[...] + da_ref[...] + db_ref[...] + 1e-16
    xr = (n0_ref[...] + pa_ref[...] + pb_ref[...]) / den[:, None] + b_ref[...]
    xr = jnp.maximum(xr, 0.0)
    h = jnp.dot(xr, w_ref[...], preferred_element_type=jnp.float32)
    hp = jnp.concatenate([h, jnp.zeros((PN - N, D), jnp.float32)], axis=0)
    p, q, w0 = _attn_tail(hp, asrc_ref[...], adst_ref[...])
    h_ref[...] = hp
    p_ref[...] = p
    q_ref[...] = q
    n0o_ref[...] = h * w0[:, None]
    d0o_ref[...] = w0


def _fin_body(n0_ref, pa_ref, pb_ref, d0_ref, da_ref, db_ref, b_ref, out_ref):
    den = d0_ref[...] + da_ref[...] + db_ref[...] + 1e-16
    out_ref[...] = (n0_ref[...] + pa_ref[...] + pb_ref[...]) / den[:, None] \
        + b_ref[...]


_OUT_MATP = jax.ShapeDtypeStruct((PN, D), jnp.float32)
_OUT_VECP = jax.ShapeDtypeStruct((PN,), jnp.float32)
_OUT_MAT = jax.ShapeDtypeStruct((N, D), jnp.float32)
_OUT_VEC = jax.ShapeDtypeStruct((N,), jnp.float32)


def _tc_pre(x, W, a_src, a_dst):
    return pl.pallas_call(
        _pre_body,
        out_shape=[_OUT_MATP, _OUT_VECP, _OUT_VECP, _OUT_MAT, _OUT_VEC],
    )(x, W, a_src.reshape(1, D), a_dst.reshape(1, D))


def _tc_mid(n0, pa, pb, d0, da, db, b, W, a_src, a_dst):
    return pl.pallas_call(
        _mid_body,
        out_shape=[_OUT_MATP, _OUT_VECP, _OUT_VECP, _OUT_MAT, _OUT_VEC],
    )(n0, pa, pb, d0, da, db, b.reshape(1, D), W,
      a_src.reshape(1, D), a_dst.reshape(1, D))


def _tc_fin(n0, pa, pb, d0, da, db, b):
    return pl.pallas_call(
        _fin_body,
        out_shape=_OUT_MAT,
    )(n0, pa, pb, d0, da, db, b.reshape(1, D))


_MESH = plsc.VectorSubcoreMesh(
    core_axis_name="c", subcore_axis_name="s", num_cores=NC, num_subcores=NS)


@functools.partial(
    pl.kernel,
    out_type=[jax.ShapeDtypeStruct((NC * NP, D), jnp.float32),
              jax.ShapeDtypeStruct((NC * NP,), jnp.float32)],
    mesh=_MESH,
    compiler_params=pltpu.CompilerParams(needs_layout_passes=False),
    scratch_types=[
        pltpu.VMEM((PN,), jnp.float32),
        pltpu.VMEM((PN,), jnp.float32),
        pltpu.VMEM((K, D), jnp.float32),
        pltpu.VMEM((K, D), jnp.float32),
        pltpu.VMEM((K,), jnp.int32),
        pltpu.VMEM((K,), jnp.int32),
        pltpu.VMEM((K,), jnp.int32),
        pltpu.VMEM((K,), jnp.int32),
        pltpu.VMEM((K,), jnp.float32),
        pltpu.VMEM((K,), jnp.float32),
        pltpu.VMEM_SHARED((NP, D), jnp.float32),
        pltpu.VMEM_SHARED((NP,), jnp.float32),
        pltpu.SemaphoreType.DMA,
        pltpu.SemaphoreType.DMA,
        pltpu.SemaphoreType.DMA,
        pltpu.SemaphoreType.DMA,
        pltpu.SemaphoreType.DMA,
        pltpu.SemaphoreType.DMA,
        pltpu.SemaphoreType.DMA,
        pltpu.SemaphoreType.DMA,
        pltpu.SemaphoreType.DMA,
        pltpu.SemaphoreType.DMA,
        pltpu.SemaphoreType.DMA,
        pltpu.SemaphoreType.DMA,
    ],
)
def _sc_edge(srcA_hbm, srcB_hbm, dstA_hbm, dstB_hbm, h_hbm, p_hbm, q_hbm,
             z_hbm, zn_hbm, parts_hbm, dparts_hbm,
             p_loc, q_loc, rows0, rows1, src_c0, src_c1, dst_c0, dst_c1,
             w_c0, w_c1, numer_sh, den_sh, semr0, semr1, semsrc0, semsrc1,
             semi0, semi1, sems0, sems1, semd0, semd1, semz, semzn):
    c = lax.axis_index("c")
    s = lax.axis_index("s")
    wid = c * NS + s

    pltpu.async_copy(z_hbm.at[pl.ds(s * RPT, RPT)],
                     numer_sh.at[pl.ds(s * RPT, RPT)], semz)
    @pl.when(s == 0)
    def _():
        pltpu.async_copy(zn_hbm, den_sh, semzn)
    pltpu.sync_copy(p_hbm, p_loc)
    pltpu.sync_copy(q_hbm, q_loc)

    def idx_view(arrA, arrB, ci, odd):
        g = (wid * NCHUNK + ci) * K
        if odd:
            return arrB.at[pl.ds(g - K, K)]
        return arrA.at[pl.ds(g, K)]

    def fetch_src(ci, odd, src_c, semsrc):
        pltpu.async_copy(idx_view(srcA_hbm, srcB_hbm, ci, odd), src_c,
                         semsrc)

    def fetch_dst(ci, odd, dst_c, semi):
        pltpu.async_copy(idx_view(dstA_hbm, dstB_hbm, ci, odd), dst_c, semi)

    def start_gather(ci, odd, rows, src_c, semr, semsrc):
        pltpu.make_async_copy(idx_view(srcA_hbm, srcB_hbm, ci, odd), src_c,
                              semsrc).wait()
        pltpu.async_copy(h_hbm.at[src_c], rows, semr)

    def work_chunk(ci, odd, rows, src_c, dst_c, w_c,
                   semr, semsrc, semi, sems, semd):
        pltpu.make_async_copy(idx_view(dstA_hbm, dstB_hbm, ci, odd), dst_c,
                              semi).wait()

        def w_body(j, carry):
            sv = src_c[pl.ds(j * 16, 16)]
            dv = dst_c[pl.ds(j * 16, 16)]
            tt = plsc.load_gather(p_loc, [sv]) + plsc.load_gather(q_loc, [dv])
            tt = jnp.where(tt >= 0.0, tt, NEG * tt)
            w_c[pl.ds(j * 16, 16)] = jnp.exp(tt)
            return carry
        lax.fori_loop(0, K // 16, w_body, 0)

        pltpu.make_async_copy(h_hbm.at[src_c], rows, semr).wait()

        @pl.when(ci + 2 < NCHUNK)
        def _():
            fetch_src(ci + 2, odd, src_c, semsrc)

        def r_body(e, carry):
            wb = plsc.load_gather(w_c, [jnp.full((16,), e, jnp.int32)])
            for j in range(D // 16):
                rows[e, pl.ds(j * 16, 16)] = rows[e, pl.ds(j * 16, 16)] * wb
            return carry
        lax.fori_loop(0, K, r_body, 0)

        pltpu.async_copy(rows, numer_sh.at[dst_c], sems, add=True)
        pltpu.async_copy(w_c, den_sh.at[dst_c], semd, add=True)

    def drain_chunk(rows, dst_c, w_c, sems, semd):
        pltpu.make_async_copy(rows, numer_sh.at[dst_c], sems).wait()
        pltpu.make_async_copy(w_c, den_sh.at[dst_c], semd).wait()

    fetch_src(0, False, src_c0, semsrc0)
    fetch_dst(0, False, dst_c0, semi0)
    fetch_src(1, True, src_c1, semsrc1)
    fetch_dst(1, True, dst_c1, semi1)
    start_gather(0, False, rows0, src_c0, semr0, semsrc0)
    start_gather(1, True, rows1, src_c1, semr1, semsrc1)

    pltpu.make_async_copy(z_hbm.at[pl.ds(s * RPT, RPT)],
                          numer_sh.at[pl.ds(s * RPT, RPT)], semz).wait()
    @pl.when(s == 0)
    def _():
        pltpu.make_async_copy(zn_hbm, den_sh, semzn).wait()
    plsc.subcore_barrier()

    def pipe_body(i, carry):
        ci = 2 * i
        work_chunk(ci, False, rows0, src_c0, dst_c0, w_c0,
                   semr0, semsrc0, semi0, sems0, semd0)
        work_chunk(ci + 1, True, rows1, src_c1, dst_c1, w_c1,
                   semr1, semsrc1, semi1, sems1, semd1)
        drain_chunk(rows0, dst_c0, w_c0, sems0, semd0)
        @pl.when(ci + 2 < NCHUNK)
        def _():
            fetch_dst(ci + 2, False, dst_c0, semi0)
            start_gather(ci + 2, False, rows0, src_c0, semr0, semsrc0)
        drain_chunk(rows1, dst_c1, w_c1, sems1, semd1)
        @pl.when(ci + 3 < NCHUNK)
        def _():
            fetch_dst(ci + 3, True, dst_c1, semi1)
            start_gather(ci + 3, True, rows1, src_c1, semr1, semsrc1)
        return carry
    lax.fori_loop(0, NCHUNK // 2, pipe_body, 0)

    plsc.subcore_barrier()
    pltpu.sync_copy(numer_sh.at[pl.ds(s * RPT, RPT)],
                    parts_hbm.at[pl.ds(c * NP + s * RPT, RPT)])
    @pl.when(s == 0)
    def _():
        pltpu.sync_copy(den_sh, dparts_hbm.at[pl.ds(c * NP, NP)])


def kernel(x, edge_index, W1, a_src1, a_dst1, b1, W2, a_src2, a_dst2, b2):
    pad = jnp.full((EP - E,), N, jnp.int32)
    padK = jnp.full((K,), N, jnp.int32)
    padd = N + jnp.arange(EP - E, dtype=jnp.int32) % (NP - N)
    srcA = jnp.concatenate([edge_index[0], pad])
    srcB = jnp.concatenate([srcA[K:], padK])
    dstA = jnp.concatenate([edge_index[1], padd])
    dstB = jnp.concatenate([dstA[K:], padK])
    zrow = jnp.zeros((NP, D), jnp.float32)
    zn = jnp.zeros((NP,), jnp.float32)

    h1, p1, q1, n01, d01 = _tc_pre(x, W1, a_src1, a_dst1)
    parts1, dparts1 = _sc_edge(srcA, srcB, dstA, dstB, h1, p1, q1, zrow, zn)
    h2, p2, q2, n02, d02 = _tc_mid(
        n01, parts1[:N], parts1[NP:NP + N], d01, dparts1[:N],
        dparts1[NP:NP + N], b1, W2, a_src2, a_dst2)
    parts2, dparts2 = _sc_edge(srcA, srcB, dstA, dstB, h2, p2, q2, zrow, zn)
    out = _tc_fin(n02, parts2[:N], parts2[NP:NP + N], d02, dparts2[:N],
                  dparts2[NP:NP + N], b2)
    return out
